# Initial kernel scaffold; baseline (speedup 1.0000x reference)
import jax
import jax.numpy as jnp
from jax.experimental import pallas as pl


def kernel(support_tensors, support_labels_name, overwrite):
    raise NotImplementedError("write your pallas kernel here")


# speedup vs baseline: 3.3063x; 3.3063x over previous
import jax
import jax.numpy as jnp
from jax import lax
from jax.experimental import pallas as pl
from jax.experimental.pallas import tpu as pltpu
from jax.experimental.pallas import tpu_sc as plsc

_N = 100000
_NV = 256
_L = 16
_NC = 2
_NS = 16
_NW = _NC * _NS

_P1 = 6256
_P1_SKIP = (_P1 * _NS - _N) // _L

_W = 3128
_C = 128
_T3 = (_W + _C - 1) // _C


def _sc_body(lab_hbm, out_hbm, m_hbm,
             lab_v, hist, parts_sh, parts_v, counts_v, rank_v, inv_v,
             block, idx_v, rows_v, sem):
    cid = lax.axis_index("c")
    sid = lax.axis_index("s")
    wid = sid * _NC + cid

    zi = jnp.zeros((_L,), jnp.int32)
    zf = jnp.zeros((_L,), jnp.float32)
    ones = jnp.ones((_L,), jnp.int32)
    lane = lax.iota(jnp.int32, _L)

    base1 = jnp.minimum(sid * _P1, _N - _P1)
    pltpu.sync_copy(lab_hbm.at[pl.ds(base1, _P1)], lab_v)

    def zero_hist(i, c):
        hist[pl.ds(i * _L, _L)] = zi
        return c
    lax.fori_loop(0, (_L * _NV) // _L, zero_hist, 0)

    lane_off = lane * _NV

    def hist_step(j, c):
        v = lab_v[pl.ds(j * _L, _L)]
        plsc.addupdate_scatter(hist, [lane_off + v], ones)
        return c
    j0 = jnp.where(sid == _NS - 1, _P1_SKIP, 0)
    lax.fori_loop(j0, _P1 // _L, hist_step, 0)

    def fold_step(k, c):
        acc = zi
        for l in range(_L):
            acc = acc + hist[pl.ds(l * _NV + k * _L, _L)]
        counts_v[pl.ds(k * _L, _L)] = acc
        return c
    lax.fori_loop(0, _NV // _L, fold_step, 0)

    pltpu.sync_copy(counts_v, parts_sh.at[sid])
    plsc.subcore_barrier()
    pltpu.sync_copy(parts_sh, parts_v)

    def total_step(k, c):
        acc = zi
        for l in range(_NS):
            acc = acc + parts_v[l, pl.ds(k * _L, _L)]
        counts_v[pl.ds(k * _L, _L)] = acc
        return c
    lax.fori_loop(0, _NV // _L, total_step, 0)

    def scan_step(k, carry):
        cvec = counts_v[pl.ds(k * _L, _L)]
        pres = cvec > 0
        pres_i = jnp.where(pres, 1, 0).astype(jnp.int32)
        incl = plsc.cumsum(pres_i)
        rank_vec = incl - pres_i + carry

        @pl.when(k == sid)
        def _():
            rank_v[...] = rank_vec
            inv_v[...] = jnp.where(pres, 1.0 / cvec.astype(jnp.float32), 0.0)

        return carry + jnp.sum(pres_i)
    lax.fori_loop(0, sid + 1, scan_step, jnp.int32(0))

    for r in range(_L):
        for k2 in range(_NV // _L):
            block[r, pl.ds(k2 * _L, _L)] = zf
    plsc.store_scatter(block, [lane, rank_v[...]], inv_v[...])
    pltpu.sync_copy(block, m_hbm.at[pl.ds(sid * _L, _L), :])
    plsc.subcore_barrier()

    base3 = jnp.minimum(wid * _W, _N - _W)

    def gather_step(t, c):
        off = base3 + jnp.minimum(t * _C, _W - _C)
        pltpu.sync_copy(lab_hbm.at[pl.ds(off, _C)], idx_v)
        pltpu.async_copy(m_hbm.at[idx_v], rows_v, sem).wait()
        pltpu.sync_copy(rows_v, out_hbm.at[pl.ds(off, _C), :])
        return c
    lax.fori_loop(0, _T3, gather_step, 0)


def _sc_onehot(labels):
    mesh = plsc.VectorSubcoreMesh(core_axis_name="c", subcore_axis_name="s")
    f = pl.kernel(
        _sc_body,
        out_type=[
            jax.ShapeDtypeStruct((_N, _NV), jnp.float32),
            jax.ShapeDtypeStruct((_NV, _NV), jnp.float32),
        ],
        mesh=mesh,
        compiler_params=pltpu.CompilerParams(needs_layout_passes=False),
        scratch_types=[
            pltpu.VMEM((_P1,), jnp.int32),
            pltpu.VMEM((_L * _NV,), jnp.int32),
            pltpu.VMEM_SHARED((_NS, _NV), jnp.int32),
            pltpu.VMEM((_NS, _NV), jnp.int32),
            pltpu.VMEM((_NV,), jnp.int32),
            pltpu.VMEM((_L,), jnp.int32),
            pltpu.VMEM((_L,), jnp.float32),
            pltpu.VMEM((_L, _NV), jnp.float32),
            pltpu.VMEM((_C,), jnp.int32),
            pltpu.VMEM((_C, _NV), jnp.float32),
            pltpu.SemaphoreType.DMA,
        ],
    )
    out, _m = f(labels)
    return out


def kernel(support_tensors, support_labels_name, overwrite):
    labels = support_labels_name.astype(jnp.int32)
    one_hot = _sc_onehot(labels)
    loss = jnp.zeros((1,), jnp.float32)
    return support_tensors, one_hot, loss
